# baseline (speedup 1.0000x reference)
import jax
import jax.numpy as jnp
import numpy as np
from jax.experimental import pallas as pl
from jax.experimental.pallas import tpu as pltpu

_EPS = 1e-5
_INV = 1.0 / np.sqrt(1.0 + _EPS)


def _dot_bf16(a, b):
    return jax.lax.dot_general(
        a, b, dimension_numbers=(((1,), (0,)), ((), ())),
        preferred_element_type=jnp.float32)


def _dot_f32(a, b):
    return jax.lax.dot_general(
        a, b, dimension_numbers=(((1,), (0,)), ((), ())),
        precision=jax.lax.Precision.HIGHEST,
        preferred_element_type=jnp.float32)


def _layer1_body(adj_ref, x_ref, W1_ref, b1_ref, g_ref, be_ref, W2_ref,
                 out_ref, s1_ref):
    i = pl.program_id(0)

    @pl.when(i == 0)
    def _():
        s1_ref[...] = _dot_f32(x_ref[...], W1_ref[...]).astype(jnp.bfloat16)

    acc = _dot_bf16(adj_ref[...].astype(jnp.bfloat16), s1_ref[...])
    y = g_ref[...] * ((acc + b1_ref[...]) * _INV) + be_ref[...]
    y = jnp.maximum(y, 0.0)
    out_ref[...] = _dot_f32(y, W2_ref[...]).astype(jnp.bfloat16)


def _layer2_body(adj_ref, s2_ref, b2_ref, g_ref, be_ref,
                 m1W_ref, m1b_ref, m1g_ref, m1be_ref,
                 m2W_ref, m2b_ref, m2g_ref, m2be_ref,
                 m3W_ref, m3b_ref, out_ref):
    acc = _dot_bf16(adj_ref[...].astype(jnp.bfloat16), s2_ref[...])
    y = g_ref[...] * ((acc + b2_ref[...]) * _INV) + be_ref[...]
    y = jnp.maximum(y, 0.0)
    h = _dot_f32(y, m1W_ref[...]) + m1b_ref[...]
    h = jnp.maximum(m1g_ref[...] * (h * _INV) + m1be_ref[...], 0.0)
    h = _dot_f32(h, m2W_ref[...]) + m2b_ref[...]
    h = jnp.maximum(m2g_ref[...] * (h * _INV) + m2be_ref[...], 0.0)
    lo = _dot_f32(h, m3W_ref[...]) + m3b_ref[...]
    m = jnp.max(lo, axis=-1, keepdims=True)
    lse = jnp.log(jnp.sum(jnp.exp(lo - m), axis=-1, keepdims=True)) + m
    out_ref[...] = lo - lse


def kernel(x, adj, W1, b1, W2, b2, bn1_g, bn1_b, m1_W, m1_b, m1_g, m1_be,
           m2_W, m2_b, m2_g, m2_be, m3_W, m3_b):
    N, nfeat = x.shape
    nhid = W1.shape[1]
    nmid = m1_W.shape[1]
    nclass = m3_W.shape[1]
    BM = 400
    grid = (N // BM,)

    def row(r):
        return r.reshape(1, -1)

    def const_spec(shape):
        return pl.BlockSpec(shape, lambda i: (0, 0))

    cparams = pltpu.CompilerParams(vmem_limit_bytes=100 * 1024 * 1024)

    s2 = pl.pallas_call(
        _layer1_body,
        grid=grid,
        in_specs=[
            pl.BlockSpec((BM, N), lambda i: (i, 0)),
            const_spec((N, nfeat)),
            const_spec((nfeat, nhid)),
            const_spec((1, nhid)),
            const_spec((1, nhid)),
            const_spec((1, nhid)),
            const_spec((nhid, nhid)),
        ],
        out_specs=pl.BlockSpec((BM, nhid), lambda i: (i, 0)),
        out_shape=jax.ShapeDtypeStruct((N, nhid), jnp.bfloat16),
        scratch_shapes=[pltpu.VMEM((N, nhid), jnp.bfloat16)],
        compiler_params=cparams,
    )(adj, x, W1, row(b1), row(bn1_g), row(bn1_b), W2)

    out = pl.pallas_call(
        _layer2_body,
        grid=grid,
        in_specs=[
            pl.BlockSpec((BM, N), lambda i: (i, 0)),
            const_spec((N, nhid)),
            const_spec((1, nhid)),
            const_spec((1, nhid)),
            const_spec((1, nhid)),
            const_spec((nhid, nmid)),
            const_spec((1, nmid)),
            const_spec((1, nmid)),
            const_spec((1, nmid)),
            const_spec((nmid, nhid)),
            const_spec((1, nhid)),
            const_spec((1, nhid)),
            const_spec((1, nhid)),
            const_spec((nhid, nclass)),
            const_spec((1, nclass)),
        ],
        out_specs=pl.BlockSpec((BM, nclass), lambda i: (i, 0)),
        out_shape=jax.ShapeDtypeStruct((N, nclass), jnp.float32),
        compiler_params=cparams,
    )(adj, s2, row(b2), row(bn1_g), row(bn1_b),
      m1_W, row(m1_b), row(m1_g), row(m1_be),
      m2_W, row(m2_b), row(m2_g), row(m2_be),
      m3_W, row(m3_b))
    return out
